# baseline (speedup 1.0000x reference)
import functools

import jax
import jax.numpy as jnp
from jax import lax
from jax.experimental import pallas as pl
from jax.experimental.pallas import tpu as pltpu
from jax.experimental.pallas import tpu_sc as plsc

D_MODEL = 64
SCALE = 8.0

NC, NS = 2, 16
NW = NC * NS
XROWS, XCOLS = 4096, 200
RPW = XROWS // NW
NBUF = 4
LOOKAHEAD = 2
F = XCOLS * D_MODEL

_mesh = plsc.VectorSubcoreMesh(core_axis_name="c", subcore_axis_name="s")


@functools.partial(
    pl.kernel,
    out_type=jax.ShapeDtypeStruct((XROWS, XCOLS, D_MODEL), jnp.float32),
    mesh=_mesh,
    scratch_types=[
        pltpu.VMEM((RPW, XCOLS), jnp.int32),
        [pltpu.VMEM((XCOLS, D_MODEL), jnp.float32)] * NBUF,
        [pltpu.SemaphoreType.DMA] * NBUF,
        [pltpu.SemaphoreType.DMA] * NBUF,
    ],
    compiler_params=pltpu.CompilerParams(use_tc_tiling_on_sc=False),
)
def _gather_sc(x_hbm, tab_hbm, out_hbm, idx_v, rows, gsems, osems):
    wid = lax.axis_index("s") * NC + lax.axis_index("c")
    base = wid * RPW

    pltpu.sync_copy(x_hbm.at[pl.ds(base, RPW)], idx_v)

    def issue_gather(j, b):
        pltpu.async_copy(tab_hbm.at[idx_v.at[j]], rows[b], gsems[b])

    def wait_gather(b):
        pltpu.make_async_copy(tab_hbm.at[idx_v.at[0]], rows[b], gsems[b]).wait()

    def wait_out(b):
        pltpu.make_async_copy(rows[b], out_hbm.at[0], osems[b]).wait()

    def consume(j, b):
        wait_gather(b)
        pltpu.async_copy(rows[b], out_hbm.at[base + j], osems[b])

    def visit(j, b, drain_out):
        bf = (b + LOOKAHEAD) % NBUF
        if drain_out:
            wait_out(bf)
        issue_gather(j + LOOKAHEAD, bf)
        consume(j, b)

    for b in range(LOOKAHEAD):
        issue_gather(b, b)
    for j in range(NBUF):
        visit(j, j, drain_out=(j >= NBUF - LOOKAHEAD))

    def steady(g, carry):
        for b in range(NBUF):
            visit(g * NBUF + b, b, drain_out=True)
        return carry

    lax.fori_loop(1, RPW // NBUF - 1, steady, 0)

    for b in range(NBUF):
        j = RPW - NBUF + b
        if b < NBUF - LOOKAHEAD:
            visit(j, b, drain_out=True)
        else:
            consume(j, b)
    for b in range(NBUF):
        wait_out(b)


TB, TF = 256, 512


def _tscale_body(a_ref, o_ref):
    o_ref[...] = a_ref[...].T * SCALE


_tscale = pl.pallas_call(
    _tscale_body,
    grid=(F // TF, XROWS // TB),
    in_specs=[pl.BlockSpec((TB, TF), lambda f, b: (b, f))],
    out_specs=pl.BlockSpec((TF, TB), lambda f, b: (f, b)),
    out_shape=jax.ShapeDtypeStruct((F, XROWS), jnp.float32),
)


def kernel(x, table):
    gathered = _gather_sc(x.astype(jnp.int32), table)
    out_t = _tscale(gathered.reshape(XROWS, F))
    return out_t.reshape(XCOLS, D_MODEL, XROWS).transpose(2, 0, 1)


# speedup vs baseline: 1.4009x; 1.4009x over previous
import functools

import jax
import jax.numpy as jnp
from jax import lax
from jax.experimental import pallas as pl
from jax.experimental.pallas import tpu as pltpu
from jax.experimental.pallas import tpu_sc as plsc

D_MODEL = 64
SCALE = 8.0

NC, NS, LANES = 2, 16, 16
NW = NC * NS
XROWS, XCOLS = 4096, 200
B = XROWS * XCOLS
RPW = XROWS // NW
JB = D_MODEL // LANES
NBUF = 8
LOOKAHEAD = 4

_mesh = plsc.VectorSubcoreMesh(core_axis_name="c", subcore_axis_name="s")


@functools.partial(
    pl.kernel,
    out_type=jax.ShapeDtypeStruct((B, D_MODEL), jnp.float32),
    mesh=_mesh,
    scratch_types=[
        pltpu.VMEM((RPW, XCOLS), jnp.int32),
        [pltpu.VMEM((XCOLS, D_MODEL), jnp.float32)] * NBUF,
        [pltpu.SemaphoreType.DMA] * NBUF,
        [pltpu.SemaphoreType.DMA] * NBUF,
    ],
    compiler_params=pltpu.CompilerParams(use_tc_tiling_on_sc=False),
)
def _embed_sc(x_hbm, tab_hbm, out_hbm, idx_v, rows, gsems, osems):
    wid = lax.axis_index("s") * NC + lax.axis_index("c")
    base = wid * RPW * XCOLS

    pltpu.sync_copy(x_hbm.at[pl.ds(wid * RPW, RPW)], idx_v)

    def issue_gather(j, b):
        pltpu.async_copy(tab_hbm.at[idx_v.at[j]], rows[b], gsems[b])

    def wait_gather(b):
        pltpu.make_async_copy(tab_hbm.at[idx_v.at[0]], rows[b], gsems[b]).wait()

    def wait_out(b):
        pltpu.make_async_copy(rows[b], out_hbm.at[pl.ds(0, XCOLS)], osems[b]).wait()

    def consume(j, b):
        wait_gather(b)

        def scale_row(r, carry):
            for k in range(JB):
                v = rows[b][r, pl.ds(k * LANES, LANES)] * SCALE
                rows[b][r, pl.ds(k * LANES, LANES)] = v
            return carry

        lax.fori_loop(0, XCOLS, scale_row, 0)
        pltpu.async_copy(
            rows[b], out_hbm.at[pl.ds(base + j * XCOLS, XCOLS)], osems[b]
        )

    def visit(j, b, drain_out):
        bf = (b + LOOKAHEAD) % NBUF
        if drain_out:
            wait_out(bf)
        issue_gather(j + LOOKAHEAD, bf)
        consume(j, b)

    for b in range(LOOKAHEAD):
        issue_gather(b, b)
    for j in range(NBUF):
        visit(j, j, drain_out=(j >= NBUF - LOOKAHEAD))

    def steady(g, carry):
        for b in range(NBUF):
            visit(g * NBUF + b, b, drain_out=True)
        return carry

    lax.fori_loop(1, RPW // NBUF - 1, steady, 0)

    for b in range(NBUF):
        j = RPW - NBUF + b
        if b < NBUF - LOOKAHEAD:
            visit(j, b, drain_out=True)
        else:
            consume(j, b)
    for b in range(NBUF):
        wait_out(b)


def kernel(x, table):
    out_flat = _embed_sc(x.astype(jnp.int32), table)
    return out_flat.reshape(XROWS, XCOLS, D_MODEL)
